# Initial kernel scaffold; baseline (speedup 1.0000x reference)
import jax
import jax.numpy as jnp
from jax.experimental import pallas as pl


def kernel(x, freqs_complex, start_pos, attn_norm_w, ffn_norm_w, wq, wk, wv, wo, router_w, w1, w2):
    raise NotImplementedError("write your pallas kernel here")


# speedup vs baseline: 1.1424x; 1.1424x over previous
import jax
import jax.numpy as jnp
from jax.experimental import pallas as pl
from jax.experimental.pallas import tpu as pltpu

D = 768
NH = 12
DH = 64
NE = 8
TK = 2
DHID = 1536
S = 2048
AEPS = 1e-6
FEPS = 1e-6

BR = 256
BQ = 256
BR2 = 1024
BH = 512


def _qkv_body(x_ref, wa_ref, wb_ref, cos_ref, sin_ref, nw_ref, o_ref):
    j = pl.program_id(0)
    x = x_ref[...]
    xn = x * jax.lax.rsqrt(jnp.mean(x * x, axis=1, keepdims=True) + AEPS) * nw_ref[...]
    a = jnp.dot(xn, wa_ref[0], preferred_element_type=jnp.float32)

    @pl.when(j < 2)
    def _():
        b = jnp.dot(xn, wb_ref[0], preferred_element_type=jnp.float32)
        o_ref[0] = a * cos_ref[...] + b * sin_ref[...]

    @pl.when(j == 2)
    def _():
        o_ref[0] = a


def _attn_body(sp_ref, q_ref, k_ref, v_ref, o_ref):
    qb = pl.program_id(1)
    q = q_ref[0]
    k = k_ref[0]
    s = jax.lax.dot_general(q, k, (((1,), (1,)), ((), ())),
                            preferred_element_type=jnp.float32) * 0.125
    rows = qb * BQ + jax.lax.broadcasted_iota(jnp.int32, (BQ, S), 0) + sp_ref[0]
    cols = jax.lax.broadcasted_iota(jnp.int32, (BQ, S), 1)
    s = jnp.where(cols <= rows, s, -1e9)
    m = jnp.max(s, axis=1, keepdims=True)
    p = jnp.exp(s - m)
    l = jnp.sum(p, axis=1, keepdims=True)
    o_ref[0] = jnp.dot(p, v_ref[0], preferred_element_type=jnp.float32) / l


def _post_body(attn_ref, x_ref, wo_ref, nw_ref, rw_ref,
               h_ref, hn_ref, g_ref, i_ref, acc_ref):
    r = pl.program_id(0)
    h = x_ref[...] + jnp.dot(attn_ref[...], wo_ref[...],
                             preferred_element_type=jnp.float32)
    h_ref[...] = h
    hn = h * jax.lax.rsqrt(jnp.mean(h * h, axis=1, keepdims=True) + FEPS) * nw_ref[...]
    hn_ref[...] = hn
    logits = jnp.dot(hn, rw_ref[...], preferred_element_type=jnp.float32)
    lane = jax.lax.broadcasted_iota(jnp.int32, (BR, 128), 1)
    valid = lane < NE
    logits = jnp.where(valid, logits, -jnp.inf)
    m = jnp.max(logits, axis=1, keepdims=True)
    e = jnp.exp(logits - m)
    probs = e / jnp.sum(e, axis=1, keepdims=True)
    v1 = jnp.max(probs, axis=1, keepdims=True)
    i1 = jnp.min(jnp.where(probs == v1, lane, NE), axis=1, keepdims=True)
    p2 = jnp.where(lane == i1, -1.0, probs)
    v2 = jnp.max(p2, axis=1, keepdims=True)
    i2 = jnp.min(jnp.where(p2 == v2, lane, NE), axis=1, keepdims=True)
    gs = v1 + v2
    col0 = lane == 0
    col1 = lane == 1
    g_ref[...] = jnp.where(col0, v1 / gs, 0.0) + jnp.where(col1, v2 / gs, 0.0)
    i_ref[...] = jnp.where(col0, i1, 0) + jnp.where(col1, i2, 0)

    @pl.when(r == 0)
    def _():
        acc_ref[...] = jnp.zeros_like(acc_ref)

    psum = jnp.sum(probs, axis=0, keepdims=True)
    dsum = jnp.sum((lane == i1).astype(jnp.float32)
                   + (lane == i2).astype(jnp.float32), axis=0, keepdims=True)
    acc_ref[0:1, :] += psum
    acc_ref[1:2, :] += dsum


def _moe_body(hn_ref, h_ref, g_ref, i_ref, w1_ref, w2_ref, o_ref):
    e = pl.program_id(1)
    hh = pl.program_id(2)

    @pl.when((e == 0) & (hh == 0))
    def _():
        o_ref[...] = h_ref[...]

    t = jnp.maximum(jnp.dot(hn_ref[...], w1_ref[0],
                            preferred_element_type=jnp.float32), 0.0)
    y = jnp.dot(t, w2_ref[0], preferred_element_type=jnp.float32)
    g1 = g_ref[:, 0:1]
    g2 = g_ref[:, 1:2]
    i1 = i_ref[:, 0:1]
    i2 = i_ref[:, 1:2]
    ge = jnp.where(i1 == e, g1, 0.0) + jnp.where(i2 == e, g2, 0.0)
    o_ref[...] += ge * y


def _swapw(w):
    wr = w.reshape(D, D // 2, 2)
    return jnp.stack([-wr[:, :, 1], wr[:, :, 0]], axis=-1).reshape(D, D)


def kernel(x, freqs_complex, start_pos, attn_norm_w, ffn_norm_w,
           wq, wk, wv, wo, router_w, w1, w2):
    xf = x.reshape(S, D)
    cos_t = jnp.cos(freqs_complex)
    sin_t = jnp.sin(freqs_complex)
    cosI = jnp.tile(jnp.repeat(cos_t, 2, axis=1), (1, NH))
    sinI = jnp.tile(jnp.repeat(sin_t, 2, axis=1), (1, NH))
    wa = jnp.stack([wq, wk, wv])
    wb = jnp.stack([_swapw(wq), _swapw(wk)])
    anw = attn_norm_w.reshape(1, D)
    fnw = ffn_norm_w.reshape(1, D)
    rwp = jnp.pad(router_w, ((0, 0), (0, 128 - NE)))
    sp = jnp.asarray(start_pos, jnp.int32).reshape(1)

    z3 = pl.pallas_call(
        _qkv_body,
        grid=(3, S // BR),
        in_specs=[
            pl.BlockSpec((BR, D), lambda j, r: (r, 0)),
            pl.BlockSpec((1, D, D), lambda j, r: (j, 0, 0)),
            pl.BlockSpec((1, D, D), lambda j, r: (jnp.minimum(j, 1), 0, 0)),
            pl.BlockSpec((BR, D), lambda j, r: (r, 0)),
            pl.BlockSpec((BR, D), lambda j, r: (r, 0)),
            pl.BlockSpec((1, D), lambda j, r: (0, 0)),
        ],
        out_specs=pl.BlockSpec((1, BR, D), lambda j, r: (j, r, 0)),
        out_shape=jax.ShapeDtypeStruct((3, S, D), jnp.float32),
    )(xf, wa, wb, cosI, sinI, anw)

    q3 = z3[0].reshape(S, NH, DH).transpose(1, 0, 2)
    k3 = z3[1].reshape(S, NH, DH).transpose(1, 0, 2)
    v3 = z3[2].reshape(S, NH, DH).transpose(1, 0, 2)

    attn3 = pl.pallas_call(
        _attn_body,
        grid=(NH, S // BQ),
        in_specs=[
            pl.BlockSpec(memory_space=pltpu.SMEM),
            pl.BlockSpec((1, BQ, DH), lambda h, qb: (h, qb, 0)),
            pl.BlockSpec((1, S, DH), lambda h, qb: (h, 0, 0)),
            pl.BlockSpec((1, S, DH), lambda h, qb: (h, 0, 0)),
        ],
        out_specs=pl.BlockSpec((1, BQ, DH), lambda h, qb: (h, qb, 0)),
        out_shape=jax.ShapeDtypeStruct((NH, S, DH), jnp.float32),
    )(sp, q3, k3, v3)

    attn = attn3.transpose(1, 0, 2).reshape(S, D)

    h, hn, g, idx, acc = pl.pallas_call(
        _post_body,
        grid=(S // BR,),
        in_specs=[
            pl.BlockSpec((BR, D), lambda r: (r, 0)),
            pl.BlockSpec((BR, D), lambda r: (r, 0)),
            pl.BlockSpec((D, D), lambda r: (0, 0)),
            pl.BlockSpec((1, D), lambda r: (0, 0)),
            pl.BlockSpec((D, 128), lambda r: (0, 0)),
        ],
        out_specs=[
            pl.BlockSpec((BR, D), lambda r: (r, 0)),
            pl.BlockSpec((BR, D), lambda r: (r, 0)),
            pl.BlockSpec((BR, 128), lambda r: (r, 0)),
            pl.BlockSpec((BR, 128), lambda r: (r, 0)),
            pl.BlockSpec((8, 128), lambda r: (0, 0)),
        ],
        out_shape=[
            jax.ShapeDtypeStruct((S, D), jnp.float32),
            jax.ShapeDtypeStruct((S, D), jnp.float32),
            jax.ShapeDtypeStruct((S, 128), jnp.float32),
            jax.ShapeDtypeStruct((S, 128), jnp.int32),
            jax.ShapeDtypeStruct((8, 128), jnp.float32),
        ],
    )(attn, xf, wo, fnw, rwp)

    out = pl.pallas_call(
        _moe_body,
        grid=(S // BR2, NE, DHID // BH),
        in_specs=[
            pl.BlockSpec((BR2, D), lambda r, e, hh: (r, 0)),
            pl.BlockSpec((BR2, D), lambda r, e, hh: (r, 0)),
            pl.BlockSpec((BR2, 128), lambda r, e, hh: (r, 0)),
            pl.BlockSpec((BR2, 128), lambda r, e, hh: (r, 0)),
            pl.BlockSpec((1, D, BH), lambda r, e, hh: (e, 0, hh)),
            pl.BlockSpec((1, BH, D), lambda r, e, hh: (e, hh, 0)),
        ],
        out_specs=pl.BlockSpec((BR2, D), lambda r, e, hh: (r, 0)),
        out_shape=jax.ShapeDtypeStruct((S, D), jnp.float32),
    )(hn, h, g, idx, w1, w2)

    f = acc[1, :NE] / (S * TK)
    p = acc[0, :NE] / S
    loss = NE * jnp.sum(f * p)
    return (out.reshape(1, S, D), loss)


# speedup vs baseline: 1.1953x; 1.0463x over previous
import jax
import jax.numpy as jnp
from jax.experimental import pallas as pl
from jax.experimental.pallas import tpu as pltpu

D = 768
NH = 12
DH = 64
NE = 8
TK = 2
DHID = 1536
S = 2048
AEPS = 1e-6
FEPS = 1e-6

BR = 256
BQ = 256
BR2 = 2048
BH = 512


def _qkv_body(x_ref, wa_ref, wb_ref, cos_ref, sin_ref, nw_ref, o_ref):
    j = pl.program_id(0)
    x = x_ref[...]
    xn = x * jax.lax.rsqrt(jnp.mean(x * x, axis=1, keepdims=True) + AEPS) * nw_ref[...]
    xnb = xn.astype(jnp.bfloat16)
    a = jnp.dot(xnb, wa_ref[0], preferred_element_type=jnp.float32)

    @pl.when(j < 2)
    def _():
        b = jnp.dot(xnb, wb_ref[0], preferred_element_type=jnp.float32)
        o_ref[0] = (a * cos_ref[...] + b * sin_ref[...]).astype(jnp.bfloat16)

    @pl.when(j == 2)
    def _():
        o_ref[0] = a.astype(jnp.bfloat16)


def _attn_body(sp_ref, q_ref, k_ref, v_ref, o_ref):
    qb = pl.program_id(1)
    q = q_ref[0]
    k = k_ref[0]
    s = jax.lax.dot_general(q, k, (((1,), (1,)), ((), ())),
                            preferred_element_type=jnp.float32) * 0.125
    rows = qb * BQ + jax.lax.broadcasted_iota(jnp.int32, (BQ, S), 0) + sp_ref[0]
    cols = jax.lax.broadcasted_iota(jnp.int32, (BQ, S), 1)
    s = jnp.where(cols <= rows, s, -1e9)
    m = jnp.max(s, axis=1, keepdims=True)
    p = jnp.exp(s - m)
    l = jnp.sum(p, axis=1, keepdims=True)
    pv = jnp.dot(p.astype(jnp.bfloat16), v_ref[0],
                 preferred_element_type=jnp.float32)
    o_ref[0] = (pv / l).astype(jnp.bfloat16)


def _post_body(attn_ref, x_ref, wo_ref, nw_ref, rw_ref,
               h_ref, hn_ref, g_ref, i_ref, acc_ref):
    r = pl.program_id(0)
    h = x_ref[...] + jnp.dot(attn_ref[...], wo_ref[...],
                             preferred_element_type=jnp.float32)
    h_ref[...] = h
    hn = h * jax.lax.rsqrt(jnp.mean(h * h, axis=1, keepdims=True) + FEPS) * nw_ref[...]
    hn_ref[...] = hn.astype(jnp.bfloat16)
    logits = jnp.dot(hn, rw_ref[...], preferred_element_type=jnp.float32)
    lane = jax.lax.broadcasted_iota(jnp.int32, (BR, 128), 1)
    valid = lane < NE
    logits = jnp.where(valid, logits, -jnp.inf)
    m = jnp.max(logits, axis=1, keepdims=True)
    e = jnp.exp(logits - m)
    probs = e / jnp.sum(e, axis=1, keepdims=True)
    v1 = jnp.max(probs, axis=1, keepdims=True)
    i1 = jnp.min(jnp.where(probs == v1, lane, NE), axis=1, keepdims=True)
    p2 = jnp.where(lane == i1, -1.0, probs)
    v2 = jnp.max(p2, axis=1, keepdims=True)
    i2 = jnp.min(jnp.where(p2 == v2, lane, NE), axis=1, keepdims=True)
    gs = v1 + v2
    col0 = lane == 0
    col1 = lane == 1
    g_ref[...] = jnp.where(col0, v1 / gs, 0.0) + jnp.where(col1, v2 / gs, 0.0)
    i_ref[...] = jnp.where(col0, i1, 0) + jnp.where(col1, i2, 0)

    @pl.when(r == 0)
    def _():
        acc_ref[...] = jnp.zeros_like(acc_ref)

    psum = jnp.sum(probs, axis=0, keepdims=True)
    dsum = jnp.sum((lane == i1).astype(jnp.float32)
                   + (lane == i2).astype(jnp.float32), axis=0, keepdims=True)
    acc_ref[0:1, :] += psum
    acc_ref[1:2, :] += dsum


def _moe_body(hn_ref, h_ref, g_ref, i_ref, w1_ref, w2_ref, o_ref):
    e = pl.program_id(1)
    hh = pl.program_id(2)

    @pl.when((e == 0) & (hh == 0))
    def _():
        o_ref[...] = h_ref[...]

    t = jnp.maximum(jnp.dot(hn_ref[...], w1_ref[0],
                            preferred_element_type=jnp.float32), 0.0)
    y = jnp.dot(t.astype(jnp.bfloat16), w2_ref[0],
                preferred_element_type=jnp.float32)
    g1 = g_ref[:, 0:1]
    g2 = g_ref[:, 1:2]
    i1 = i_ref[:, 0:1]
    i2 = i_ref[:, 1:2]
    ge = jnp.where(i1 == e, g1, 0.0) + jnp.where(i2 == e, g2, 0.0)
    o_ref[...] += ge * y


def _swapw(w):
    wr = w.reshape(D, D // 2, 2)
    return jnp.stack([-wr[:, :, 1], wr[:, :, 0]], axis=-1).reshape(D, D)


def kernel(x, freqs_complex, start_pos, attn_norm_w, ffn_norm_w,
           wq, wk, wv, wo, router_w, w1, w2):
    xf = x.reshape(S, D)
    cos_t = jnp.cos(freqs_complex)
    sin_t = jnp.sin(freqs_complex)
    cosI = jnp.tile(jnp.repeat(cos_t, 2, axis=1), (1, NH))
    sinI = jnp.tile(jnp.repeat(sin_t, 2, axis=1), (1, NH))
    wa = jnp.stack([wq, wk, wv]).astype(jnp.bfloat16)
    wb = jnp.stack([_swapw(wq), _swapw(wk)]).astype(jnp.bfloat16)
    wob = wo.astype(jnp.bfloat16)
    w1b = w1.astype(jnp.bfloat16)
    w2b = w2.astype(jnp.bfloat16)
    anw = attn_norm_w.reshape(1, D)
    fnw = ffn_norm_w.reshape(1, D)
    rwp = jnp.pad(router_w, ((0, 0), (0, 128 - NE)))
    sp = jnp.asarray(start_pos, jnp.int32).reshape(1)

    z3 = pl.pallas_call(
        _qkv_body,
        grid=(3, S // BR),
        in_specs=[
            pl.BlockSpec((BR, D), lambda j, r: (r, 0)),
            pl.BlockSpec((1, D, D), lambda j, r: (j, 0, 0)),
            pl.BlockSpec((1, D, D), lambda j, r: (jnp.minimum(j, 1), 0, 0)),
            pl.BlockSpec((BR, D), lambda j, r: (r, 0)),
            pl.BlockSpec((BR, D), lambda j, r: (r, 0)),
            pl.BlockSpec((1, D), lambda j, r: (0, 0)),
        ],
        out_specs=pl.BlockSpec((1, BR, D), lambda j, r: (j, r, 0)),
        out_shape=jax.ShapeDtypeStruct((3, S, D), jnp.bfloat16),
    )(xf, wa, wb, cosI, sinI, anw)

    q3 = z3[0].reshape(S, NH, DH).transpose(1, 0, 2)
    k3 = z3[1].reshape(S, NH, DH).transpose(1, 0, 2)
    v3 = z3[2].reshape(S, NH, DH).transpose(1, 0, 2)

    attn3 = pl.pallas_call(
        _attn_body,
        grid=(NH, S // BQ),
        in_specs=[
            pl.BlockSpec(memory_space=pltpu.SMEM),
            pl.BlockSpec((1, BQ, DH), lambda h, qb: (h, qb, 0)),
            pl.BlockSpec((1, S, DH), lambda h, qb: (h, 0, 0)),
            pl.BlockSpec((1, S, DH), lambda h, qb: (h, 0, 0)),
        ],
        out_specs=pl.BlockSpec((1, BQ, DH), lambda h, qb: (h, qb, 0)),
        out_shape=jax.ShapeDtypeStruct((NH, S, DH), jnp.bfloat16),
    )(sp, q3, k3, v3)

    attn = attn3.transpose(1, 0, 2).reshape(S, D)

    h, hn, g, idx, acc = pl.pallas_call(
        _post_body,
        grid=(S // BR,),
        in_specs=[
            pl.BlockSpec((BR, D), lambda r: (r, 0)),
            pl.BlockSpec((BR, D), lambda r: (r, 0)),
            pl.BlockSpec((D, D), lambda r: (0, 0)),
            pl.BlockSpec((1, D), lambda r: (0, 0)),
            pl.BlockSpec((D, 128), lambda r: (0, 0)),
        ],
        out_specs=[
            pl.BlockSpec((BR, D), lambda r: (r, 0)),
            pl.BlockSpec((BR, D), lambda r: (r, 0)),
            pl.BlockSpec((BR, 128), lambda r: (r, 0)),
            pl.BlockSpec((BR, 128), lambda r: (r, 0)),
            pl.BlockSpec((8, 128), lambda r: (0, 0)),
        ],
        out_shape=[
            jax.ShapeDtypeStruct((S, D), jnp.float32),
            jax.ShapeDtypeStruct((S, D), jnp.bfloat16),
            jax.ShapeDtypeStruct((S, 128), jnp.float32),
            jax.ShapeDtypeStruct((S, 128), jnp.int32),
            jax.ShapeDtypeStruct((8, 128), jnp.float32),
        ],
    )(attn, xf, wob, fnw, rwp)

    out = pl.pallas_call(
        _moe_body,
        grid=(S // BR2, NE, DHID // BH),
        in_specs=[
            pl.BlockSpec((BR2, D), lambda r, e, hh: (r, 0)),
            pl.BlockSpec((BR2, D), lambda r, e, hh: (r, 0)),
            pl.BlockSpec((BR2, 128), lambda r, e, hh: (r, 0)),
            pl.BlockSpec((BR2, 128), lambda r, e, hh: (r, 0)),
            pl.BlockSpec((1, D, BH), lambda r, e, hh: (e, 0, hh)),
            pl.BlockSpec((1, BH, D), lambda r, e, hh: (e, hh, 0)),
        ],
        out_specs=pl.BlockSpec((BR2, D), lambda r, e, hh: (r, 0)),
        out_shape=jax.ShapeDtypeStruct((S, D), jnp.float32),
    )(hn, h, g, idx, w1b, w2b)

    f = acc[1, :NE] / (S * TK)
    p = acc[0, :NE] / S
    loss = NE * jnp.sum(f * p)
    return (out.reshape(1, S, D), loss)
